# baseline (speedup 1.0000x reference)
import jax
import jax.numpy as jnp
from jax import lax
from jax.experimental import pallas as pl
from jax.experimental.pallas import tpu as pltpu
from jax.experimental.pallas import tpu_sc as plsc

N = 10000
E = 320000
D = 128
HALF = D // 2

NC = 2
NS = 16
NW = NC * NS
K = 128
SCH = 160
DCH = 80
EPAD = NW * DCH * K
PH = 20
NPH = SCH // PH
NB = 5
STRIPE = N // NS
AR = N + 8

HR = 10240
HSTR = HR // NS

_mesh_cache = []


def _mesh():
    if not _mesh_cache:
        _mesh_cache.append(plsc.VectorSubcoreMesh(
            core_axis_name="c", subcore_axis_name="s",
            num_cores=NC, num_subcores=NS))
    return _mesh_cache[0]


def _fill_1d(ref, nwords, value):
    def body(i, carry):
        ref[pl.ds(i * 16, 16)] = jnp.full((16,), value, jnp.float32)
        return carry

    lax.fori_loop(0, nwords // 16, body, 0)


def _deg_body(dst_hbm, out_hbm, dstv, onesv, stagev, deg_sh):
    c = lax.axis_index("c")
    s = lax.axis_index("s")
    pltpu.sync_copy(dst_hbm.at[s, pl.ds(c * DCH, DCH)], dstv)
    _fill_1d(onesv, K, 1.0)
    _fill_1d(stagev, HSTR, 0.0)
    pltpu.sync_copy(stagev, deg_sh.at[pl.ds(s * HSTR, HSTR)])
    plsc.subcore_barrier()

    def body(j, carry):
        pltpu.sync_copy(onesv, deg_sh.at[dstv.at[j]], add=True)
        return carry

    lax.fori_loop(0, DCH, body, 0)
    plsc.subcore_barrier()
    pltpu.sync_copy(deg_sh.at[pl.ds(s * HSTR, HSTR)], stagev)
    pltpu.sync_copy(stagev, out_hbm.at[pl.ds(c * HR + s * HSTR, HSTR)])


def _sc_degree(dst16):
    return pl.kernel(
        _deg_body,
        out_type=jax.ShapeDtypeStruct((NC * HR,), jnp.float32),
        mesh=_mesh(),
        compiler_params=pltpu.CompilerParams(use_tc_tiling_on_sc=False),
        scratch_types=[
            pltpu.VMEM((DCH, K), jnp.int32),
            pltpu.VMEM((K,), jnp.float32),
            pltpu.VMEM((HSTR,), jnp.float32),
            pltpu.VMEM_SHARED((HR,), jnp.float32),
        ],
    )(dst16)


def _scatter_body(hs0_hbm, hs1_hbm, src_hbm, dst_hbm, out0_hbm, out1_hbm,
                  srcv, dstv, rows_a, rows_b, rows_c, rows_d, rows_e,
                  acc_sh, tbl_sh,
                  gsem_a, gsem_b, gsem_c, gsem_d, gsem_e,
                  ssem_a, ssem_b, ssem_c, ssem_d, ssem_e):
    c = lax.axis_index("c")
    s = lax.axis_index("s")
    rows = [rows_a, rows_b, rows_c, rows_d, rows_e]
    gsem = [gsem_a, gsem_b, gsem_c, gsem_d, gsem_e]
    ssem = [ssem_a, ssem_b, ssem_c, ssem_d, ssem_e]

    def zfill(i, carry):
        rows_a[lax.div(i, 4), pl.ds(lax.rem(i, 4) * 16, 16)] = (
            jnp.zeros((16,), jnp.float32))
        return carry

    lax.fori_loop(0, K * 4, zfill, 0)

    def zcopy(i, carry):
        pltpu.sync_copy(rows_a.at[pl.ds(0, STRIPE // 5)],
                        acc_sh.at[pl.ds(s * STRIPE + i * (STRIPE // 5),
                                        STRIPE // 5)])
        return carry

    lax.fori_loop(0, 5, zcopy, 0)

    @pl.when(s == NS - 1)
    def _():
        pltpu.sync_copy(rows_a.at[pl.ds(0, 8)], acc_sh.at[pl.ds(N, 8)])

    def pipeline(hs_hbm):
        pltpu.sync_copy(hs_hbm.at[pl.ds(s * STRIPE, STRIPE)],
                        tbl_sh.at[pl.ds(s * STRIPE, STRIPE)])
        plsc.subcore_barrier()

        def phase(p, carry):
            pltpu.sync_copy(src_hbm.at[s, pl.ds(p * PH, PH)], srcv)
            pltpu.sync_copy(dst_hbm.at[s, pl.ds(p * PH, PH)], dstv)

            for b in range(NB):
                pltpu.async_copy(tbl_sh.at[srcv.at[b]], rows[b], gsem[b])

            def body(t, carry2):
                j0 = t * NB
                for b in range(NB):
                    pltpu.make_async_copy(
                        tbl_sh.at[srcv.at[0]], rows[b], gsem[b]).wait()
                for b in range(NB):
                    pltpu.async_copy(
                        rows[b], acc_sh.at[dstv.at[j0 + b]], ssem[b], add=True)
                for b in range(NB):
                    pltpu.make_async_copy(
                        rows[b], acc_sh.at[dstv.at[0]], ssem[b]).wait()
                    jn = lax.rem(j0 + NB + b, PH)
                    pltpu.async_copy(tbl_sh.at[srcv.at[jn]], rows[b], gsem[b])
                return carry2

            lax.fori_loop(0, PH // NB, body, 0)
            for b in range(NB):
                pltpu.make_async_copy(
                    tbl_sh.at[srcv.at[0]], rows[b], gsem[b]).wait()
            return carry

        lax.fori_loop(0, NPH, phase, 0)

    @pl.when(c == 0)
    def _():
        pipeline(hs0_hbm)

    @pl.when(c == 1)
    def _():
        pipeline(hs1_hbm)

    plsc.subcore_barrier()

    @pl.when(c == 0)
    def _():
        pltpu.sync_copy(acc_sh.at[pl.ds(s * STRIPE, STRIPE)],
                        out0_hbm.at[pl.ds(s * STRIPE, STRIPE)])

    @pl.when(c == 1)
    def _():
        pltpu.sync_copy(acc_sh.at[pl.ds(s * STRIPE, STRIPE)],
                        out1_hbm.at[pl.ds(s * STRIPE, STRIPE)])


def _sc_scatter(hs0, hs1, src16, dst16):
    return pl.kernel(
        _scatter_body,
        out_type=(jax.ShapeDtypeStruct((N, HALF), jnp.float32),
                  jax.ShapeDtypeStruct((N, HALF), jnp.float32)),
        mesh=_mesh(),
        compiler_params=pltpu.CompilerParams(use_tc_tiling_on_sc=False),
        scratch_types=[
            pltpu.VMEM((PH, K), jnp.int32),
            pltpu.VMEM((PH, K), jnp.int32),
            pltpu.VMEM((K, HALF), jnp.float32),
            pltpu.VMEM((K, HALF), jnp.float32),
            pltpu.VMEM((K, HALF), jnp.float32),
            pltpu.VMEM((K, HALF), jnp.float32),
            pltpu.VMEM((K, HALF), jnp.float32),
            pltpu.VMEM_SHARED((AR, HALF), jnp.float32),
            pltpu.VMEM_SHARED((N, HALF), jnp.float32),
            pltpu.SemaphoreType.DMA,
            pltpu.SemaphoreType.DMA,
            pltpu.SemaphoreType.DMA,
            pltpu.SemaphoreType.DMA,
            pltpu.SemaphoreType.DMA,
            pltpu.SemaphoreType.DMA,
            pltpu.SemaphoreType.DMA,
            pltpu.SemaphoreType.DMA,
            pltpu.SemaphoreType.DMA,
            pltpu.SemaphoreType.DMA,
        ],
    )(hs0, hs1, src16, dst16)


_BR = 2000


def _row_spec(w):
    return pl.BlockSpec((_BR, w), lambda i: (i, 0))


def _full_spec(h, w):
    return pl.BlockSpec((h, w), lambda i: (0, 0))


def _dinv_body(dp_ref, o_ref):
    o_ref[...] = lax.rsqrt(1.0 + dp_ref[0] + dp_ref[1])


def _tc_dinv(deg_flat):
    dp = deg_flat.reshape(NC, HR // 128, 128)
    return pl.pallas_call(
        _dinv_body,
        out_shape=jax.ShapeDtypeStruct((HR // 128, 128), jnp.float32),
    )(dp)


def _mm1_body(x_ref, w_ref, dinv_ref, o0_ref, o1_ref):
    h = jnp.dot(x_ref[...], w_ref[...], preferred_element_type=jnp.float32)
    hs = h * dinv_ref[...]
    o0_ref[...] = hs[:, 0:HALF]
    o1_ref[...] = hs[:, HALF:D]


def _tc_layer1(x, w1, dinv_col):
    return pl.pallas_call(
        _mm1_body,
        grid=(N // _BR,),
        in_specs=[_row_spec(D), _full_spec(D, D), _row_spec(1)],
        out_specs=(_row_spec(HALF), _row_spec(HALF)),
        out_shape=(jax.ShapeDtypeStruct((N, HALF), jnp.float32),
                   jax.ShapeDtypeStruct((N, HALF), jnp.float32)),
    )(x, w1, dinv_col)


def _mm2_body(p0_ref, p1_ref, hs0_ref, hs1_ref, dinv_ref, b_ref, w_ref,
              o0_ref, o1_ref):
    acc = jnp.concatenate(
        [p0_ref[...] + hs0_ref[...], p1_ref[...] + hs1_ref[...]], axis=1)
    z = jnp.maximum(acc * dinv_ref[...] + b_ref[...], 0.0)
    h = jnp.dot(z, w_ref[...], preferred_element_type=jnp.float32)
    hs = h * dinv_ref[...]
    o0_ref[...] = hs[:, 0:HALF]
    o1_ref[...] = hs[:, HALF:D]


def _tc_layer2(p0, p1, hs0, hs1, dinv_col, b1, w2):
    return pl.pallas_call(
        _mm2_body,
        grid=(N // _BR,),
        in_specs=[_row_spec(HALF), _row_spec(HALF), _row_spec(HALF),
                  _row_spec(HALF), _row_spec(1), _full_spec(1, D),
                  _full_spec(D, D)],
        out_specs=(_row_spec(HALF), _row_spec(HALF)),
        out_shape=(jax.ShapeDtypeStruct((N, HALF), jnp.float32),
                   jax.ShapeDtypeStruct((N, HALF), jnp.float32)),
    )(p0, p1, hs0, hs1, dinv_col, b1, w2)


def _fin_body(p0_ref, p1_ref, hs0_ref, hs1_ref, dinv_ref, b_ref, o_ref):
    acc = jnp.concatenate(
        [p0_ref[...] + hs0_ref[...], p1_ref[...] + hs1_ref[...]], axis=1)
    o_ref[...] = acc * dinv_ref[...] + b_ref[...]


def _tc_final(p0, p1, hs0, hs1, dinv_col, b2):
    return pl.pallas_call(
        _fin_body,
        grid=(N // _BR,),
        in_specs=[_row_spec(HALF), _row_spec(HALF), _row_spec(HALF),
                  _row_spec(HALF), _row_spec(1), _full_spec(1, D)],
        out_specs=_row_spec(D),
        out_shape=jax.ShapeDtypeStruct((N, D), jnp.float32),
    )(p0, p1, hs0, hs1, dinv_col, b2)


def kernel(x, edge_index, W1, b1, W2, b2):
    pad = EPAD - E
    src16 = jnp.concatenate(
        [edge_index[0], jnp.zeros((pad,), jnp.int32)]).reshape(NS, SCH, K)
    dst16 = jnp.concatenate(
        [edge_index[1], jnp.full((pad,), N, jnp.int32)]).reshape(NS, SCH, K)

    deg_flat = _sc_degree(dst16)
    dinv_pk = _tc_dinv(deg_flat)
    dinv_col = dinv_pk.reshape(HR)[:N].reshape(N, 1)

    hs1_0, hs1_1 = _tc_layer1(x, W1, dinv_col)
    p1_0, p1_1 = _sc_scatter(hs1_0, hs1_1, src16, dst16)
    hs2_0, hs2_1 = _tc_layer2(p1_0, p1_1, hs1_0, hs1_1, dinv_col,
                              b1.reshape(1, D), W2)
    p2_0, p2_1 = _sc_scatter(hs2_0, hs2_1, src16, dst16)
    out = _tc_final(p2_0, p2_1, hs2_0, hs2_1, dinv_col, b2.reshape(1, D))
    return out


# speedup vs baseline: 1.1471x; 1.1471x over previous
import jax
import jax.numpy as jnp
from jax import lax
from jax.experimental import pallas as pl
from jax.experimental.pallas import tpu as pltpu
from jax.experimental.pallas import tpu_sc as plsc

N = 10000
E = 320000
D = 128
HALF = D // 2

NC = 2
NS = 16
NW = NC * NS
K = 125
SCH = 160
DCH = 80
PH = 40
NPH = SCH // PH
NB = 5
STRIPE = N // NS

HR = 10240
HSTR = HR // NS

_mesh_cache = []


def _mesh():
    if not _mesh_cache:
        _mesh_cache.append(plsc.VectorSubcoreMesh(
            core_axis_name="c", subcore_axis_name="s",
            num_cores=NC, num_subcores=NS))
    return _mesh_cache[0]


def _fill_1d(ref, nwords, value):
    def body(i, carry):
        ref[pl.ds(i * 16, 16)] = jnp.full((16,), value, jnp.float32)
        return carry

    lax.fori_loop(0, nwords // 16, body, 0)


def _deg_body(dst_hbm, out_hbm, dstv, onesv, stagev, deg_sh):
    c = lax.axis_index("c")
    s = lax.axis_index("s")
    pltpu.sync_copy(dst_hbm.at[s, pl.ds(c * DCH, DCH)], dstv)
    _fill_1d(onesv, 128, 1.0)
    _fill_1d(stagev, HSTR, 0.0)
    pltpu.sync_copy(stagev, deg_sh.at[pl.ds(s * HSTR, HSTR)])
    plsc.subcore_barrier()

    def body(j, carry):
        pltpu.sync_copy(onesv.at[pl.ds(0, K)], deg_sh.at[dstv.at[j]],
                        add=True)
        return carry

    lax.fori_loop(0, DCH, body, 0)
    plsc.subcore_barrier()
    pltpu.sync_copy(deg_sh.at[pl.ds(s * HSTR, HSTR)], stagev)
    pltpu.sync_copy(stagev, out_hbm.at[pl.ds(c * HR + s * HSTR, HSTR)])


def _sc_degree(dst16):
    return pl.kernel(
        _deg_body,
        out_type=jax.ShapeDtypeStruct((NC * HR,), jnp.float32),
        mesh=_mesh(),
        compiler_params=pltpu.CompilerParams(use_tc_tiling_on_sc=False),
        scratch_types=[
            pltpu.VMEM((DCH, K), jnp.int32),
            pltpu.VMEM((128,), jnp.float32),
            pltpu.VMEM((HSTR,), jnp.float32),
            pltpu.VMEM_SHARED((HR,), jnp.float32),
        ],
    )(dst16)


def _scatter_body(hs0_hbm, hs1_hbm, src_hbm, dst_hbm, out0_hbm, out1_hbm,
                  srcv, dstv, rows_a, rows_b, rows_c, rows_d, rows_e,
                  acc_sh, tbl_sh,
                  gsem_a, gsem_b, gsem_c, gsem_d, gsem_e,
                  ssem_a, ssem_b, ssem_c, ssem_d, ssem_e):
    c = lax.axis_index("c")
    s = lax.axis_index("s")
    rows = [rows_a, rows_b, rows_c, rows_d, rows_e]
    gsem = [gsem_a, gsem_b, gsem_c, gsem_d, gsem_e]
    ssem = [ssem_a, ssem_b, ssem_c, ssem_d, ssem_e]

    def zfill(i, carry):
        rows_a[lax.div(i, 4), pl.ds(lax.rem(i, 4) * 16, 16)] = (
            jnp.zeros((16,), jnp.float32))
        return carry

    lax.fori_loop(0, K * 4, zfill, 0)

    def zcopy(i, carry):
        pltpu.sync_copy(rows_a, acc_sh.at[pl.ds(s * STRIPE + i * K, K)])
        return carry

    lax.fori_loop(0, STRIPE // K, zcopy, 0)

    def pipeline(hs_hbm):
        pltpu.sync_copy(hs_hbm.at[pl.ds(s * STRIPE, STRIPE)],
                        tbl_sh.at[pl.ds(s * STRIPE, STRIPE)])
        plsc.subcore_barrier()

        def phase(p, carry):
            pltpu.sync_copy(src_hbm.at[s, pl.ds(p * PH, PH)], srcv)
            pltpu.sync_copy(dst_hbm.at[s, pl.ds(p * PH, PH)], dstv)

            for b in range(NB):
                pltpu.async_copy(tbl_sh.at[srcv.at[b]], rows[b], gsem[b])

            def body(t, carry2):
                j0 = t * NB
                for b in range(NB):
                    pltpu.make_async_copy(
                        tbl_sh.at[srcv.at[0]], rows[b], gsem[b]).wait()
                for b in range(NB):
                    pltpu.async_copy(
                        rows[b], acc_sh.at[dstv.at[j0 + b]], ssem[b], add=True)
                for b in range(NB):
                    pltpu.make_async_copy(
                        rows[b], acc_sh.at[dstv.at[0]], ssem[b]).wait()
                    jn = lax.rem(j0 + NB + b, PH)
                    pltpu.async_copy(tbl_sh.at[srcv.at[jn]], rows[b], gsem[b])
                return carry2

            lax.fori_loop(0, PH // NB, body, 0)
            for b in range(NB):
                pltpu.make_async_copy(
                    tbl_sh.at[srcv.at[0]], rows[b], gsem[b]).wait()
            return carry

        lax.fori_loop(0, NPH, phase, 0)

    @pl.when(c == 0)
    def _():
        pipeline(hs0_hbm)

    @pl.when(c == 1)
    def _():
        pipeline(hs1_hbm)

    plsc.subcore_barrier()

    @pl.when(c == 0)
    def _():
        pltpu.sync_copy(acc_sh.at[pl.ds(s * STRIPE, STRIPE)],
                        out0_hbm.at[pl.ds(s * STRIPE, STRIPE)])

    @pl.when(c == 1)
    def _():
        pltpu.sync_copy(acc_sh.at[pl.ds(s * STRIPE, STRIPE)],
                        out1_hbm.at[pl.ds(s * STRIPE, STRIPE)])


def _sc_scatter(hs0, hs1, src16, dst16):
    return pl.kernel(
        _scatter_body,
        out_type=(jax.ShapeDtypeStruct((N, HALF), jnp.float32),
                  jax.ShapeDtypeStruct((N, HALF), jnp.float32)),
        mesh=_mesh(),
        compiler_params=pltpu.CompilerParams(use_tc_tiling_on_sc=False),
        scratch_types=[
            pltpu.VMEM((PH, K), jnp.int32),
            pltpu.VMEM((PH, K), jnp.int32),
            pltpu.VMEM((K, HALF), jnp.float32),
            pltpu.VMEM((K, HALF), jnp.float32),
            pltpu.VMEM((K, HALF), jnp.float32),
            pltpu.VMEM((K, HALF), jnp.float32),
            pltpu.VMEM((K, HALF), jnp.float32),
            pltpu.VMEM_SHARED((N, HALF), jnp.float32),
            pltpu.VMEM_SHARED((N, HALF), jnp.float32),
            pltpu.SemaphoreType.DMA,
            pltpu.SemaphoreType.DMA,
            pltpu.SemaphoreType.DMA,
            pltpu.SemaphoreType.DMA,
            pltpu.SemaphoreType.DMA,
            pltpu.SemaphoreType.DMA,
            pltpu.SemaphoreType.DMA,
            pltpu.SemaphoreType.DMA,
            pltpu.SemaphoreType.DMA,
            pltpu.SemaphoreType.DMA,
        ],
    )(hs0, hs1, src16, dst16)


_BR = 2000


def _row_spec(w):
    return pl.BlockSpec((_BR, w), lambda i: (i, 0))


def _full_spec(h, w):
    return pl.BlockSpec((h, w), lambda i: (0, 0))


def _dinv_body(dp_ref, o_ref):
    o_ref[...] = lax.rsqrt(1.0 + dp_ref[0] + dp_ref[1])


def _tc_dinv(deg_flat):
    dp = deg_flat.reshape(NC, HR // 128, 128)
    return pl.pallas_call(
        _dinv_body,
        out_shape=jax.ShapeDtypeStruct((HR // 128, 128), jnp.float32),
    )(dp)


def _mm1_body(x_ref, w_ref, dinv_ref, o0_ref, o1_ref):
    h = jnp.dot(x_ref[...], w_ref[...], preferred_element_type=jnp.float32)
    hs = h * dinv_ref[...]
    o0_ref[...] = hs[:, 0:HALF]
    o1_ref[...] = hs[:, HALF:D]


def _tc_layer1(x, w1, dinv_col):
    return pl.pallas_call(
        _mm1_body,
        grid=(N // _BR,),
        in_specs=[_row_spec(D), _full_spec(D, D), _row_spec(1)],
        out_specs=(_row_spec(HALF), _row_spec(HALF)),
        out_shape=(jax.ShapeDtypeStruct((N, HALF), jnp.float32),
                   jax.ShapeDtypeStruct((N, HALF), jnp.float32)),
    )(x, w1, dinv_col)


def _mm2_body(p0_ref, p1_ref, hs0_ref, hs1_ref, dinv_ref, b_ref, w_ref,
              o0_ref, o1_ref):
    acc = jnp.concatenate(
        [p0_ref[...] + hs0_ref[...], p1_ref[...] + hs1_ref[...]], axis=1)
    z = jnp.maximum(acc * dinv_ref[...] + b_ref[...], 0.0)
    h = jnp.dot(z, w_ref[...], preferred_element_type=jnp.float32)
    hs = h * dinv_ref[...]
    o0_ref[...] = hs[:, 0:HALF]
    o1_ref[...] = hs[:, HALF:D]


def _tc_layer2(p0, p1, hs0, hs1, dinv_col, b1, w2):
    return pl.pallas_call(
        _mm2_body,
        grid=(N // _BR,),
        in_specs=[_row_spec(HALF), _row_spec(HALF), _row_spec(HALF),
                  _row_spec(HALF), _row_spec(1), _full_spec(1, D),
                  _full_spec(D, D)],
        out_specs=(_row_spec(HALF), _row_spec(HALF)),
        out_shape=(jax.ShapeDtypeStruct((N, HALF), jnp.float32),
                   jax.ShapeDtypeStruct((N, HALF), jnp.float32)),
    )(p0, p1, hs0, hs1, dinv_col, b1, w2)


def _fin_body(p0_ref, p1_ref, hs0_ref, hs1_ref, dinv_ref, b_ref, o_ref):
    acc = jnp.concatenate(
        [p0_ref[...] + hs0_ref[...], p1_ref[...] + hs1_ref[...]], axis=1)
    o_ref[...] = acc * dinv_ref[...] + b_ref[...]


def _tc_final(p0, p1, hs0, hs1, dinv_col, b2):
    return pl.pallas_call(
        _fin_body,
        grid=(N // _BR,),
        in_specs=[_row_spec(HALF), _row_spec(HALF), _row_spec(HALF),
                  _row_spec(HALF), _row_spec(1), _full_spec(1, D)],
        out_specs=_row_spec(D),
        out_shape=jax.ShapeDtypeStruct((N, D), jnp.float32),
    )(p0, p1, hs0, hs1, dinv_col, b2)


def kernel(x, edge_index, W1, b1, W2, b2):
    src16 = edge_index[0].reshape(NS, SCH, K)
    dst16 = edge_index[1].reshape(NS, SCH, K)

    deg_flat = _sc_degree(dst16)
    dinv_pk = _tc_dinv(deg_flat)
    dinv_col = dinv_pk.reshape(HR)[:N].reshape(N, 1)

    hs1_0, hs1_1 = _tc_layer1(x, W1, dinv_col)
    p1_0, p1_1 = _sc_scatter(hs1_0, hs1_1, src16, dst16)
    hs2_0, hs2_1 = _tc_layer2(p1_0, p1_1, hs1_0, hs1_1, dinv_col,
                              b1.reshape(1, D), W2)
    p2_0, p2_1 = _sc_scatter(hs2_0, hs2_1, src16, dst16)
    out = _tc_final(p2_0, p2_1, hs2_0, hs2_1, dinv_col, b2.reshape(1, D))
    return out
